# baseline (speedup 1.0000x reference)
import functools

import jax
import jax.numpy as jnp
from jax import lax
from jax.experimental import pallas as pl
from jax.experimental.pallas import tpu as pltpu
from jax.experimental.pallas import tpu_sc as plsc

NC = 2
NS = 16
L = 16
C = 128
F32 = jnp.float32

_SC_PARAMS = pltpu.CompilerParams(use_tc_tiling_on_sc=False,
                                  needs_layout_passes=False)


def _mesh():
    return plsc.VectorSubcoreMesh(core_axis_name="c", subcore_axis_name="s",
                                  num_cores=NC, num_subcores=NS)


def _sc_gs(n_pad, d, k):
    rps = n_pad // NS

    @functools.partial(
        pl.kernel,
        out_type=jax.ShapeDtypeStruct((NC, n_pad, d), F32),
        mesh=_mesh(),
        compiler_params=_SC_PARAMS,
        scratch_types=[
            pltpu.VMEM_SHARED((n_pad, d), F32),
            pltpu.VMEM_SHARED((n_pad, d), F32),
            pltpu.VMEM((k, C), jnp.int32),
            pltpu.VMEM((k, C), jnp.int32),
            pltpu.VMEM((C, d), F32),
            pltpu.VMEM((C, d), F32),
            pltpu.SemaphoreType.DMA,
            pltpu.SemaphoreType.DMA,
        ],
    )
    def kern(idxg_hbm, idxs_hbm, table_hbm, zeros_hbm, out_hbm,
             table_s, acc_s, idxg_v, idxs_v, rows_a, rows_b, sem_a, sem_b):
        c = lax.axis_index("c")
        s = lax.axis_index("s")
        wid = c * NS + s
        r0 = s * rps
        pltpu.sync_copy(table_hbm.at[pl.ds(r0, rps)], table_s.at[pl.ds(r0, rps)])
        pltpu.sync_copy(zeros_hbm.at[pl.ds(r0, rps)], acc_s.at[pl.ds(r0, rps)])
        pltpu.sync_copy(idxg_hbm.at[pl.ds(wid * k, k)], idxg_v)
        pltpu.sync_copy(idxs_hbm.at[pl.ds(wid * k, k)], idxs_v)
        plsc.subcore_barrier()
        pltpu.async_copy(table_s.at[idxg_v.at[0]], rows_a, sem_a)

        def body(j, carry):
            jj = 2 * j
            pltpu.async_copy(table_s.at[idxg_v.at[jj + 1]], rows_b, sem_b)
            pltpu.make_async_copy(table_s.at[idxg_v.at[jj]], rows_a, sem_a).wait()
            pltpu.sync_copy(rows_a, acc_s.at[idxs_v.at[jj]], add=True)

            @pl.when(jj + 2 < k)
            def _():
                pltpu.async_copy(table_s.at[idxg_v.at[jj + 2]], rows_a, sem_a)

            pltpu.make_async_copy(table_s.at[idxg_v.at[jj + 1]], rows_b,
                                  sem_b).wait()
            pltpu.sync_copy(rows_b, acc_s.at[idxs_v.at[jj + 1]], add=True)
            return carry

        lax.fori_loop(0, k // 2, body, 0)
        plsc.subcore_barrier()
        pltpu.sync_copy(acc_s.at[pl.ds(r0, rps)], out_hbm.at[c, pl.ds(r0, rps)])

    return kern


def _sc_prep(n_pad, d, k):
    rps = n_pad // NS

    @functools.partial(
        pl.kernel,
        out_type=(jax.ShapeDtypeStruct((NC, n_pad, d), F32),
                  jax.ShapeDtypeStruct((NC, n_pad, d), F32),
                  jax.ShapeDtypeStruct((NC, n_pad, d), F32)),
        mesh=_mesh(),
        compiler_params=_SC_PARAMS,
        scratch_types=[
            pltpu.VMEM_SHARED((n_pad, d), F32),
            pltpu.VMEM_SHARED((n_pad, d), F32),
            pltpu.VMEM_SHARED((n_pad, d), F32),
            pltpu.VMEM_SHARED((n_pad, d), F32),
            pltpu.VMEM((k, C), jnp.int32),
            pltpu.VMEM((k, C), jnp.int32),
            pltpu.VMEM((C, d), F32),
            pltpu.VMEM((C, d), F32),
            pltpu.SemaphoreType.DMA,
        ],
    )
    def kern(src_hbm, dst_hbm, t_hbm, zeros_hbm, outSt_hbm, outDi_hbm, outDo_hbm,
             t_s, accSt, accDi, accDo, src_v, dst_v, gt, ones_v, sem_p):
        c = lax.axis_index("c")
        s = lax.axis_index("s")
        wid = c * NS + s
        r0 = s * rps
        pltpu.sync_copy(t_hbm.at[pl.ds(r0, rps)], t_s.at[pl.ds(r0, rps)])
        pltpu.sync_copy(zeros_hbm.at[pl.ds(r0, rps)], accSt.at[pl.ds(r0, rps)])
        pltpu.sync_copy(zeros_hbm.at[pl.ds(r0, rps)], accDi.at[pl.ds(r0, rps)])
        pltpu.sync_copy(zeros_hbm.at[pl.ds(r0, rps)], accDo.at[pl.ds(r0, rps)])
        pltpu.sync_copy(src_hbm.at[pl.ds(wid * k, k)], src_v)
        pltpu.sync_copy(dst_hbm.at[pl.ds(wid * k, k)], dst_v)
        one = jnp.ones((L,), F32)
        for i in range(C):
            ones_v[i, :] = one
        plsc.subcore_barrier()

        def body(j, carry):
            cp = pltpu.async_copy(t_s.at[src_v.at[j]], gt, sem_p)
            pltpu.sync_copy(ones_v, accDi.at[dst_v.at[j]], add=True)
            pltpu.sync_copy(ones_v, accDo.at[src_v.at[j]], add=True)
            cp.wait()
            pltpu.sync_copy(gt, accSt.at[dst_v.at[j]], add=True)
            return carry

        lax.fori_loop(0, k, body, 0)
        plsc.subcore_barrier()
        pltpu.sync_copy(accSt.at[pl.ds(r0, rps)], outSt_hbm.at[c, pl.ds(r0, rps)])
        pltpu.sync_copy(accDi.at[pl.ds(r0, rps)], outDi_hbm.at[c, pl.ds(r0, rps)])
        pltpu.sync_copy(accDo.at[pl.ds(r0, rps)], outDo_hbm.at[c, pl.ds(r0, rps)])

    return kern


def _sc_edge(n_pad, d, k):
    rps = n_pad // NS

    @functools.partial(
        pl.kernel,
        out_type=jax.ShapeDtypeStruct((NC, n_pad, d), F32),
        mesh=_mesh(),
        compiler_params=_SC_PARAMS,
        scratch_types=[
            pltpu.VMEM_SHARED((n_pad, d), F32),
            pltpu.VMEM_SHARED((n_pad, d), F32),
            pltpu.VMEM_SHARED((n_pad, d), F32),
            pltpu.VMEM((k, C), jnp.int32),
            pltpu.VMEM((k, C), jnp.int32),
            pltpu.VMEM((C, d), F32),
            pltpu.VMEM((C, d), F32),
            pltpu.VMEM((C, d), F32),
            pltpu.VMEM((C, d), F32),
            pltpu.VMEM((C, d), F32),
            pltpu.VMEM((C, d), F32),
            pltpu.SemaphoreType.DMA,
        ],
    )
    def kern(src_hbm, dst_hbm, h_hbm, m_hbm, zeros_hbm, out_hbm,
             h_s, m_s, acc_s, src_v, dst_v, ha, hb, ma, mb, pos_v, neg_v,
             sem_g):
        c = lax.axis_index("c")
        s = lax.axis_index("s")
        wid = c * NS + s
        r0 = s * rps
        pltpu.sync_copy(h_hbm.at[pl.ds(r0, rps)], h_s.at[pl.ds(r0, rps)])
        pltpu.sync_copy(m_hbm.at[pl.ds(r0, rps)], m_s.at[pl.ds(r0, rps)])
        pltpu.sync_copy(zeros_hbm.at[pl.ds(r0, rps)], acc_s.at[pl.ds(r0, rps)])
        pltpu.sync_copy(src_hbm.at[pl.ds(wid * k, k)], src_v)
        pltpu.sync_copy(dst_hbm.at[pl.ds(wid * k, k)], dst_v)
        plsc.subcore_barrier()
        iota = lax.iota(jnp.int32, L)
        shuf = [lax.bitwise_xor(iota, jnp.int32(r)) for r in (1, 2, 4, 8)]
        dnums = lax.GatherDimensionNumbers(
            offset_dims=(), collapsed_slice_dims=(0,), start_index_map=(0,))

        def hsum(v):
            for idx in shuf:
                v = v + lax.gather(
                    v, idx[:, None], dnums, (1,),
                    mode=lax.GatherScatterMode.PROMISE_IN_BOUNDS)
            return v

        def body(j, carry):
            pltpu.async_copy(h_s.at[src_v.at[j]], ha, sem_g)
            pltpu.async_copy(h_s.at[dst_v.at[j]], hb, sem_g)
            pltpu.async_copy(m_s.at[src_v.at[j]], ma, sem_g)
            pltpu.async_copy(m_s.at[dst_v.at[j]], mb, sem_g)
            pltpu.make_async_copy(h_s.at[src_v.at[j]], ha, sem_g).wait()
            pltpu.make_async_copy(h_s.at[dst_v.at[j]], hb, sem_g).wait()
            pltpu.make_async_copy(m_s.at[src_v.at[j]], ma, sem_g).wait()
            pltpu.make_async_copy(m_s.at[dst_v.at[j]], mb, sem_g).wait()

            @plsc.parallel_loop(0, C, step=1, unroll=16)
            def _edge(i):
                df = ha[i, :] - hb[i, :]
                d2 = hsum(df * df)
                cc = hsum(ma[i, :] * mb[i, :])
                yi = lax.bitcast_convert_type(
                    jnp.full((L,), 0x5F3759DF, jnp.int32)
                    - lax.shift_right_logical(
                        lax.bitcast_convert_type(d2, jnp.int32), 1),
                    F32)
                hd2 = 0.5 * d2
                yi = yi * (1.5 - hd2 * yi * yi)
                yi = yi * (1.5 - hd2 * yi * yi)
                coef = cc * yi * yi * yi
                v = coef * df
                pos_v[i, :] = v
                neg_v[i, :] = -v

            pltpu.sync_copy(pos_v, acc_s.at[dst_v.at[j]], add=True)
            pltpu.sync_copy(neg_v, acc_s.at[src_v.at[j]], add=True)
            return carry

        lax.fori_loop(0, k, body, 0)
        plsc.subcore_barrier()
        pltpu.sync_copy(acc_s.at[pl.ds(r0, rps)], out_hbm.at[c, pl.ds(r0, rps)])

    return kern


def _tc_call(body, out_shapes, *args):
    return pl.pallas_call(
        body,
        out_shape=tuple(jax.ShapeDtypeStruct(s, F32) for s in out_shapes),
    )(*args)


def _tc_pre(p128, m128):
    def body(p_ref, m_ref, t_ref):
        m = m_ref[...]
        valid = m > 0
        t_ref[...] = jnp.where(valid, p_ref[...] / jnp.where(valid, m, 1.0), 0.0)

    (t,) = _tc_call(body, [p128.shape], p128, m128)
    return t


def _tc1(accSt, accDi, accDo, q128, t128):
    def body(st_ref, di_ref, do_ref, q_ref, t_ref,
             rsin_ref, rsout_ref, x1_ref, dhdp_ref):
        rsin_ref[...] = lax.rsqrt(jnp.maximum(di_ref[0] + di_ref[1], 1.0))
        rsout_ref[...] = lax.rsqrt(jnp.maximum(do_ref[0] + do_ref[1], 1.0))
        x1_ref[...] = q_ref[...] * rsout_ref[...]
        dhdp_ref[...] = st_ref[0] + st_ref[1] + t_ref[...]

    return _tc_call(body, [q128.shape] * 4, accSt, accDi, accDo, q128, t128)


def _tc2(acc, rs_in, rs_out, W1big, b1big, W2big):
    r128, _ = rs_in.shape
    hidb = W1big.shape[1]

    def body(acc_ref, rsin_ref, rsout_ref, w1_ref, b1_ref, w2_ref,
             h2_ref, a_ref):
        xw = (acc_ref[0] + acc_ref[1]) * rsin_ref[...]
        a = jnp.dot(xw, w1_ref[...], preferred_element_type=F32) + b1_ref[...]
        a_ref[...] = a
        h2_ref[...] = jnp.dot(jnp.maximum(a, 0.0), w2_ref[...],
                              preferred_element_type=F32) * rsout_ref[...]

    return _tc_call(body, [rs_in.shape, (r128, hidb)],
                    acc, rs_in, rs_out, W1big, b1big, W2big)


def _tc3(acc, rs_in, q128, b2big):
    def body(acc_ref, rsin_ref, q_ref, b2_ref, h_ref):
        h_ref[...] = ((acc_ref[0] + acc_ref[1]) * rsin_ref[...]
                      + b2_ref[...] + q_ref[...])

    (h,) = _tc_call(body, [q128.shape], acc, rs_in, q128, b2big)
    return h


def _tc5(accG, rs_in, gravity):
    def body(acc_ref, rsin_ref, grav_ref, g_ref, y_ref):
        g = 0.5 * grav_ref[0, 0] * (acc_ref[0] + acc_ref[1])
        g_ref[...] = g
        y_ref[...] = g * rsin_ref[...]

    return _tc_call(body, [rs_in.shape] * 2, accG, rs_in, gravity)


def _tc6(acc, a_big, rs_in, rs_out, W2Tbig, W1Tbig):
    def body(acc_ref, a_ref, rsin_ref, rsout_ref, w2t_ref, w1t_ref, y2_ref):
        zw = (acc_ref[0] + acc_ref[1]) * rsout_ref[...]
        u = jnp.dot(zw, w2t_ref[...], preferred_element_type=F32)
        v = jnp.where(a_ref[...] > 0, u, 0.0)
        y2_ref[...] = jnp.dot(v, w1t_ref[...],
                              preferred_element_type=F32) * rsin_ref[...]

    (y2,) = _tc_call(body, [rs_in.shape], acc, a_big, rs_in, rs_out,
                     W2Tbig, W1Tbig)
    return y2


def _tc7(acc, rs_out, g128):
    def body(acc_ref, rsout_ref, g_ref, dhdq_ref):
        dhdq_ref[...] = (acc_ref[0] + acc_ref[1]) * rsout_ref[...] + g_ref[...]

    (dhdq,) = _tc_call(body, [rs_out.shape], acc, rs_out, g128)
    return dhdq


def kernel(q, p, edge_index, M, W1, b1, W2, b2, gravity):
    n, d = q.shape
    e = edge_index.shape[1]
    nw = NC * NS
    n_pad = -(-(n + 1) // (NS * 8)) * NS * 8
    k = -(-e // (nw * C))
    e_pad = nw * C * k
    r128 = n_pad * d // 128
    nb = 128 // d

    ei = jnp.pad(edge_index.astype(jnp.int32), ((0, 0), (0, e_pad - e)),
                 constant_values=n).reshape(2, e_pad // C, C)
    srcp, dstp = ei[0], ei[1]

    rv = n * d // 128
    rpad = ((0, r128 - rv), (0, 0))
    m = jnp.diagonal(M, axis1=1, axis2=2)
    q128 = jnp.pad(q.reshape(rv, 128), rpad)
    p128 = jnp.pad(p.reshape(rv, 128), rpad)
    m128 = jnp.pad(m.reshape(rv, 128), rpad)
    m_pad = m128.reshape(n_pad, d)
    zeros16 = jnp.zeros((n_pad, d), F32)

    eye = jnp.eye(nb, dtype=F32)
    W1big = jnp.kron(eye, W1)
    W2big = jnp.kron(eye, W2)
    W2Tbig = jnp.kron(eye, W2.T)
    W1Tbig = jnp.kron(eye, W1.T)
    b1big = jnp.tile(b1, nb).reshape(1, nb * b1.shape[0])
    b2big = jnp.tile(b2, nb).reshape(1, 128)

    def v128(acc):
        return acc.reshape(NC, r128, 128)

    def v16(x):
        return x.reshape(n_pad, d)

    sc_gs = _sc_gs(n_pad, d, k)

    t128 = _tc_pre(p128, m128)
    accSt, accDi, accDo = _sc_prep(n_pad, d, k)(srcp, dstp, v16(t128), zeros16)
    rs_in, rs_out, x1, dhdp = _tc1(v128(accSt), v128(accDi), v128(accDo),
                                   q128, t128)

    acc1 = sc_gs(srcp, dstp, v16(x1), zeros16)
    h2, a_big = _tc2(v128(acc1), rs_in, rs_out, W1big, b1big, W2big)
    acc2 = sc_gs(srcp, dstp, v16(h2), zeros16)
    h = _tc3(v128(acc2), rs_in, q128, b2big)

    accG = _sc_edge(n_pad, d, k)(srcp, dstp, v16(h), m_pad, zeros16)
    g128, y = _tc5(v128(accG), rs_in, gravity)

    accZ = sc_gs(dstp, srcp, v16(y), zeros16)
    y2 = _tc6(v128(accZ), a_big, rs_in, rs_out, W2Tbig, W1Tbig)
    accZ2 = sc_gs(dstp, srcp, v16(y2), zeros16)
    dhdq = _tc7(v128(accZ2), rs_out, g128)

    return jnp.concatenate([v16(dhdq)[:n], v16(dhdp)[:n]], axis=1)


# speedup vs baseline: 1.0562x; 1.0562x over previous
import functools

import jax
import jax.numpy as jnp
from jax import lax
from jax.experimental import pallas as pl
from jax.experimental.pallas import tpu as pltpu
from jax.experimental.pallas import tpu_sc as plsc

NC = 2
NS = 16
L = 16
C = 128
F32 = jnp.float32

_SC_PARAMS = pltpu.CompilerParams(use_tc_tiling_on_sc=False,
                                  needs_layout_passes=False)


def _mesh():
    return plsc.VectorSubcoreMesh(core_axis_name="c", subcore_axis_name="s",
                                  num_cores=NC, num_subcores=NS)


def _sc_gs(n_pad, d, k):
    rps = n_pad // NS

    @functools.partial(
        pl.kernel,
        out_type=jax.ShapeDtypeStruct((NC, n_pad, d), F32),
        mesh=_mesh(),
        compiler_params=_SC_PARAMS,
        scratch_types=[
            pltpu.VMEM_SHARED((n_pad, d), F32),
            pltpu.VMEM_SHARED((n_pad, d), F32),
            pltpu.VMEM((k, C), jnp.int32),
            pltpu.VMEM((k, C), jnp.int32),
            pltpu.VMEM((C, d), F32),
            pltpu.VMEM((C, d), F32),
            pltpu.SemaphoreType.DMA,
            pltpu.SemaphoreType.DMA,
        ],
    )
    def kern(idxg_hbm, idxs_hbm, table_hbm, zeros_hbm, out_hbm,
             table_s, acc_s, idxg_v, idxs_v, rows_a, rows_b, sem_a, sem_b):
        c = lax.axis_index("c")
        s = lax.axis_index("s")
        wid = c * NS + s
        r0 = s * rps
        pltpu.sync_copy(table_hbm.at[pl.ds(r0, rps)], table_s.at[pl.ds(r0, rps)])
        pltpu.sync_copy(zeros_hbm.at[pl.ds(r0, rps)], acc_s.at[pl.ds(r0, rps)])
        pltpu.sync_copy(idxg_hbm.at[pl.ds(wid * k, k)], idxg_v)
        pltpu.sync_copy(idxs_hbm.at[pl.ds(wid * k, k)], idxs_v)
        plsc.subcore_barrier()
        pltpu.async_copy(table_s.at[idxg_v.at[0]], rows_a, sem_a)

        def body(j, carry):
            jj = 2 * j
            pltpu.async_copy(table_s.at[idxg_v.at[jj + 1]], rows_b, sem_b)
            pltpu.make_async_copy(table_s.at[idxg_v.at[jj]], rows_a, sem_a).wait()
            pltpu.sync_copy(rows_a, acc_s.at[idxs_v.at[jj]], add=True)

            @pl.when(jj + 2 < k)
            def _():
                pltpu.async_copy(table_s.at[idxg_v.at[jj + 2]], rows_a, sem_a)

            pltpu.make_async_copy(table_s.at[idxg_v.at[jj + 1]], rows_b,
                                  sem_b).wait()
            pltpu.sync_copy(rows_b, acc_s.at[idxs_v.at[jj + 1]], add=True)
            return carry

        lax.fori_loop(0, k // 2, body, 0)
        plsc.subcore_barrier()
        pltpu.sync_copy(acc_s.at[pl.ds(r0, rps)], out_hbm.at[c, pl.ds(r0, rps)])

    return kern


def _sc_prep(n_pad, d, k):
    rps = n_pad // NS

    @functools.partial(
        pl.kernel,
        out_type=(jax.ShapeDtypeStruct((NC, n_pad, d), F32),
                  jax.ShapeDtypeStruct((NC, n_pad, d), F32),
                  jax.ShapeDtypeStruct((NC, n_pad, d), F32)),
        mesh=_mesh(),
        compiler_params=_SC_PARAMS,
        scratch_types=[
            pltpu.VMEM_SHARED((n_pad, d), F32),
            pltpu.VMEM_SHARED((n_pad, d), F32),
            pltpu.VMEM_SHARED((n_pad, d), F32),
            pltpu.VMEM_SHARED((n_pad, d), F32),
            pltpu.VMEM((k, C), jnp.int32),
            pltpu.VMEM((k, C), jnp.int32),
            pltpu.VMEM((C, d), F32),
            pltpu.VMEM((C, d), F32),
            pltpu.SemaphoreType.DMA,
        ],
    )
    def kern(src_hbm, dst_hbm, t_hbm, zeros_hbm, outSt_hbm, outDi_hbm, outDo_hbm,
             t_s, accSt, accDi, accDo, src_v, dst_v, gt, ones_v, sem_p):
        c = lax.axis_index("c")
        s = lax.axis_index("s")
        wid = c * NS + s
        r0 = s * rps
        pltpu.sync_copy(t_hbm.at[pl.ds(r0, rps)], t_s.at[pl.ds(r0, rps)])
        pltpu.sync_copy(zeros_hbm.at[pl.ds(r0, rps)], accSt.at[pl.ds(r0, rps)])
        pltpu.sync_copy(zeros_hbm.at[pl.ds(r0, rps)], accDi.at[pl.ds(r0, rps)])
        pltpu.sync_copy(zeros_hbm.at[pl.ds(r0, rps)], accDo.at[pl.ds(r0, rps)])
        pltpu.sync_copy(src_hbm.at[pl.ds(wid * k, k)], src_v)
        pltpu.sync_copy(dst_hbm.at[pl.ds(wid * k, k)], dst_v)
        one = jnp.ones((L,), F32)
        for i in range(C):
            ones_v[i, :] = one
        plsc.subcore_barrier()

        def body(j, carry):
            cp = pltpu.async_copy(t_s.at[src_v.at[j]], gt, sem_p)
            pltpu.sync_copy(ones_v, accDi.at[dst_v.at[j]], add=True)
            pltpu.sync_copy(ones_v, accDo.at[src_v.at[j]], add=True)
            cp.wait()
            pltpu.sync_copy(gt, accSt.at[dst_v.at[j]], add=True)
            return carry

        lax.fori_loop(0, k, body, 0)
        plsc.subcore_barrier()
        pltpu.sync_copy(accSt.at[pl.ds(r0, rps)], outSt_hbm.at[c, pl.ds(r0, rps)])
        pltpu.sync_copy(accDi.at[pl.ds(r0, rps)], outDi_hbm.at[c, pl.ds(r0, rps)])
        pltpu.sync_copy(accDo.at[pl.ds(r0, rps)], outDo_hbm.at[c, pl.ds(r0, rps)])

    return kern


def _sc_edge(n_pad, d, k):
    rps = n_pad // NS

    @functools.partial(
        pl.kernel,
        out_type=jax.ShapeDtypeStruct((NC, n_pad, d), F32),
        mesh=_mesh(),
        compiler_params=_SC_PARAMS,
        scratch_types=[
            pltpu.VMEM_SHARED((n_pad, d), F32),
            pltpu.VMEM_SHARED((n_pad, d), F32),
            pltpu.VMEM_SHARED((n_pad, d), F32),
            pltpu.VMEM((k, C), jnp.int32),
            pltpu.VMEM((k, C), jnp.int32),
            pltpu.VMEM((C, d), F32),
            pltpu.VMEM((C, d), F32),
            pltpu.VMEM((C, d), F32),
            pltpu.VMEM((C, d), F32),
            pltpu.VMEM((C, d), F32),
            pltpu.VMEM((C, d), F32),
            pltpu.SemaphoreType.DMA,
        ],
    )
    def kern(src_hbm, dst_hbm, h_hbm, m_hbm, zeros_hbm, out_hbm,
             h_s, m_s, acc_s, src_v, dst_v, ha, hb, ma, mb, pos_v, neg_v,
             sem_g):
        c = lax.axis_index("c")
        s = lax.axis_index("s")
        wid = c * NS + s
        r0 = s * rps
        pltpu.sync_copy(h_hbm.at[pl.ds(r0, rps)], h_s.at[pl.ds(r0, rps)])
        pltpu.sync_copy(m_hbm.at[pl.ds(r0, rps)], m_s.at[pl.ds(r0, rps)])
        pltpu.sync_copy(zeros_hbm.at[pl.ds(r0, rps)], acc_s.at[pl.ds(r0, rps)])
        pltpu.sync_copy(src_hbm.at[pl.ds(wid * k, k)], src_v)
        pltpu.sync_copy(dst_hbm.at[pl.ds(wid * k, k)], dst_v)
        plsc.subcore_barrier()
        iota = lax.iota(jnp.int32, L)
        shuf = [lax.bitwise_xor(iota, jnp.int32(r)) for r in (1, 2, 4, 8)]
        dnums = lax.GatherDimensionNumbers(
            offset_dims=(), collapsed_slice_dims=(0,), start_index_map=(0,))

        def hsum(v):
            for idx in shuf:
                v = v + lax.gather(
                    v, idx[:, None], dnums, (1,),
                    mode=lax.GatherScatterMode.PROMISE_IN_BOUNDS)
            return v

        def body(j, carry):
            pltpu.async_copy(h_s.at[src_v.at[j]], ha, sem_g)
            pltpu.async_copy(h_s.at[dst_v.at[j]], hb, sem_g)
            pltpu.async_copy(m_s.at[src_v.at[j]], ma, sem_g)
            pltpu.async_copy(m_s.at[dst_v.at[j]], mb, sem_g)
            pltpu.make_async_copy(h_s.at[src_v.at[j]], ha, sem_g).wait()
            pltpu.make_async_copy(h_s.at[dst_v.at[j]], hb, sem_g).wait()
            pltpu.make_async_copy(m_s.at[src_v.at[j]], ma, sem_g).wait()
            pltpu.make_async_copy(m_s.at[dst_v.at[j]], mb, sem_g).wait()

            @plsc.parallel_loop(0, C, step=1, unroll=8)
            def _edge(i):
                df = ha[i, :] - hb[i, :]
                d2 = hsum(df * df)
                cc = hsum(ma[i, :] * mb[i, :])
                yi = lax.bitcast_convert_type(
                    jnp.full((L,), 0x5F3759DF, jnp.int32)
                    - lax.shift_right_logical(
                        lax.bitcast_convert_type(d2, jnp.int32), 1),
                    F32)
                hd2 = 0.5 * d2
                yi = yi * (1.5 - hd2 * yi * yi)
                yi = yi * (1.5 - hd2 * yi * yi)
                coef = cc * yi * yi * yi
                v = coef * df
                pos_v[i, :] = v
                neg_v[i, :] = -v

            pltpu.sync_copy(pos_v, acc_s.at[dst_v.at[j]], add=True)
            pltpu.sync_copy(neg_v, acc_s.at[src_v.at[j]], add=True)
            return carry

        lax.fori_loop(0, k, body, 0)
        plsc.subcore_barrier()
        pltpu.sync_copy(acc_s.at[pl.ds(r0, rps)], out_hbm.at[c, pl.ds(r0, rps)])

    return kern


def _tc_call(body, out_shapes, *args):
    return pl.pallas_call(
        body,
        out_shape=tuple(jax.ShapeDtypeStruct(s, F32) for s in out_shapes),
    )(*args)


def _tc_pre(p128, m128):
    def body(p_ref, m_ref, t_ref):
        m = m_ref[...]
        valid = m > 0
        t_ref[...] = jnp.where(valid, p_ref[...] / jnp.where(valid, m, 1.0), 0.0)

    (t,) = _tc_call(body, [p128.shape], p128, m128)
    return t


def _tc1(accSt, accDi, accDo, q128, t128):
    def body(st_ref, di_ref, do_ref, q_ref, t_ref,
             rsin_ref, rsout_ref, x1_ref, dhdp_ref):
        rsin_ref[...] = lax.rsqrt(jnp.maximum(di_ref[0] + di_ref[1], 1.0))
        rsout_ref[...] = lax.rsqrt(jnp.maximum(do_ref[0] + do_ref[1], 1.0))
        x1_ref[...] = q_ref[...] * rsout_ref[...]
        dhdp_ref[...] = st_ref[0] + st_ref[1] + t_ref[...]

    return _tc_call(body, [q128.shape] * 4, accSt, accDi, accDo, q128, t128)


def _tc2(acc, rs_in, rs_out, W1big, b1big, W2big):
    r128, _ = rs_in.shape
    hidb = W1big.shape[1]

    def body(acc_ref, rsin_ref, rsout_ref, w1_ref, b1_ref, w2_ref,
             h2_ref, a_ref):
        xw = (acc_ref[0] + acc_ref[1]) * rsin_ref[...]
        a = jnp.dot(xw, w1_ref[...], preferred_element_type=F32) + b1_ref[...]
        a_ref[...] = a
        h2_ref[...] = jnp.dot(jnp.maximum(a, 0.0), w2_ref[...],
                              preferred_element_type=F32) * rsout_ref[...]

    return _tc_call(body, [rs_in.shape, (r128, hidb)],
                    acc, rs_in, rs_out, W1big, b1big, W2big)


def _tc3(acc, rs_in, q128, b2big):
    def body(acc_ref, rsin_ref, q_ref, b2_ref, h_ref):
        h_ref[...] = ((acc_ref[0] + acc_ref[1]) * rsin_ref[...]
                      + b2_ref[...] + q_ref[...])

    (h,) = _tc_call(body, [q128.shape], acc, rs_in, q128, b2big)
    return h


def _tc5(accG, rs_in, gravity):
    def body(acc_ref, rsin_ref, grav_ref, g_ref, y_ref):
        g = 0.5 * grav_ref[0, 0] * (acc_ref[0] + acc_ref[1])
        g_ref[...] = g
        y_ref[...] = g * rsin_ref[...]

    return _tc_call(body, [rs_in.shape] * 2, accG, rs_in, gravity)


def _tc6(acc, a_big, rs_in, rs_out, W2Tbig, W1Tbig):
    def body(acc_ref, a_ref, rsin_ref, rsout_ref, w2t_ref, w1t_ref, y2_ref):
        zw = (acc_ref[0] + acc_ref[1]) * rsout_ref[...]
        u = jnp.dot(zw, w2t_ref[...], preferred_element_type=F32)
        v = jnp.where(a_ref[...] > 0, u, 0.0)
        y2_ref[...] = jnp.dot(v, w1t_ref[...],
                              preferred_element_type=F32) * rsin_ref[...]

    (y2,) = _tc_call(body, [rs_in.shape], acc, a_big, rs_in, rs_out,
                     W2Tbig, W1Tbig)
    return y2


def _tc7(acc, rs_out, g128):
    def body(acc_ref, rsout_ref, g_ref, dhdq_ref):
        dhdq_ref[...] = (acc_ref[0] + acc_ref[1]) * rsout_ref[...] + g_ref[...]

    (dhdq,) = _tc_call(body, [rs_out.shape], acc, rs_out, g128)
    return dhdq


def kernel(q, p, edge_index, M, W1, b1, W2, b2, gravity):
    n, d = q.shape
    e = edge_index.shape[1]
    nw = NC * NS
    n_pad = -(-(n + 1) // (NS * 8)) * NS * 8
    k = -(-e // (nw * C))
    e_pad = nw * C * k
    r128 = n_pad * d // 128
    nb = 128 // d

    ei = jnp.pad(edge_index.astype(jnp.int32), ((0, 0), (0, e_pad - e)),
                 constant_values=n).reshape(2, e_pad // C, C)
    srcp, dstp = ei[0], ei[1]

    rv = n * d // 128
    rpad = ((0, r128 - rv), (0, 0))
    m = jnp.diagonal(M, axis1=1, axis2=2)
    q128 = jnp.pad(q.reshape(rv, 128), rpad)
    p128 = jnp.pad(p.reshape(rv, 128), rpad)
    m128 = jnp.pad(m.reshape(rv, 128), rpad)
    m_pad = m128.reshape(n_pad, d)
    zeros16 = jnp.zeros((n_pad, d), F32)

    eye = jnp.eye(nb, dtype=F32)
    W1big = jnp.kron(eye, W1)
    W2big = jnp.kron(eye, W2)
    W2Tbig = jnp.kron(eye, W2.T)
    W1Tbig = jnp.kron(eye, W1.T)
    b1big = jnp.tile(b1, nb).reshape(1, nb * b1.shape[0])
    b2big = jnp.tile(b2, nb).reshape(1, 128)

    def v128(acc):
        return acc.reshape(NC, r128, 128)

    def v16(x):
        return x.reshape(n_pad, d)

    sc_gs = _sc_gs(n_pad, d, k)

    t128 = _tc_pre(p128, m128)
    accSt, accDi, accDo = _sc_prep(n_pad, d, k)(srcp, dstp, v16(t128), zeros16)
    rs_in, rs_out, x1, dhdp = _tc1(v128(accSt), v128(accDi), v128(accDo),
                                   q128, t128)

    acc1 = sc_gs(srcp, dstp, v16(x1), zeros16)
    h2, a_big = _tc2(v128(acc1), rs_in, rs_out, W1big, b1big, W2big)
    acc2 = sc_gs(srcp, dstp, v16(h2), zeros16)
    h = _tc3(v128(acc2), rs_in, q128, b2big)

    accG = _sc_edge(n_pad, d, k)(srcp, dstp, v16(h), m_pad, zeros16)
    g128, y = _tc5(v128(accG), rs_in, gravity)

    accZ = sc_gs(dstp, srcp, v16(y), zeros16)
    y2 = _tc6(v128(accZ), a_big, rs_in, rs_out, W2Tbig, W1Tbig)
    accZ2 = sc_gs(dstp, srcp, v16(y2), zeros16)
    dhdq = _tc7(v128(accZ2), rs_out, g128)

    return jnp.concatenate([v16(dhdq)[:n], v16(dhdp)[:n]], axis=1)


# speedup vs baseline: 1.1223x; 1.0626x over previous
import functools

import jax
import jax.numpy as jnp
from jax import lax
from jax.experimental import pallas as pl
from jax.experimental.pallas import tpu as pltpu
from jax.experimental.pallas import tpu_sc as plsc

NC = 2
NS = 16
L = 16
C = 128
F32 = jnp.float32

_SC_PARAMS = pltpu.CompilerParams(use_tc_tiling_on_sc=False,
                                  needs_layout_passes=False)


def _mesh():
    return plsc.VectorSubcoreMesh(core_axis_name="c", subcore_axis_name="s",
                                  num_cores=NC, num_subcores=NS)


def _sc_gs(n_pad, d, k):
    rps = n_pad // NS

    @functools.partial(
        pl.kernel,
        out_type=jax.ShapeDtypeStruct((NC, n_pad, d), F32),
        mesh=_mesh(),
        compiler_params=_SC_PARAMS,
        scratch_types=[
            pltpu.VMEM_SHARED((n_pad, d), F32),
            pltpu.VMEM_SHARED((n_pad, d), F32),
            pltpu.VMEM((k, C), jnp.int32),
            pltpu.VMEM((k, C), jnp.int32),
            pltpu.VMEM((C, d), F32),
            pltpu.VMEM((C, d), F32),
            pltpu.SemaphoreType.DMA,
            pltpu.SemaphoreType.DMA,
        ],
    )
    def kern(idxg_hbm, idxs_hbm, table_hbm, zeros_hbm, out_hbm,
             table_s, acc_s, idxg_v, idxs_v, rows_a, rows_b, sem_a, sem_b):
        c = lax.axis_index("c")
        s = lax.axis_index("s")
        wid = c * NS + s
        r0 = s * rps
        pltpu.sync_copy(table_hbm.at[pl.ds(r0, rps)], table_s.at[pl.ds(r0, rps)])
        pltpu.sync_copy(zeros_hbm.at[pl.ds(r0, rps)], acc_s.at[pl.ds(r0, rps)])
        pltpu.sync_copy(idxg_hbm.at[pl.ds(wid * k, k)], idxg_v)
        pltpu.sync_copy(idxs_hbm.at[pl.ds(wid * k, k)], idxs_v)
        plsc.subcore_barrier()
        pltpu.async_copy(table_s.at[idxg_v.at[0]], rows_a, sem_a)

        def body(j, carry):
            jj = 2 * j
            pltpu.async_copy(table_s.at[idxg_v.at[jj + 1]], rows_b, sem_b)
            pltpu.make_async_copy(table_s.at[idxg_v.at[jj]], rows_a, sem_a).wait()
            pltpu.sync_copy(rows_a, acc_s.at[idxs_v.at[jj]], add=True)

            @pl.when(jj + 2 < k)
            def _():
                pltpu.async_copy(table_s.at[idxg_v.at[jj + 2]], rows_a, sem_a)

            pltpu.make_async_copy(table_s.at[idxg_v.at[jj + 1]], rows_b,
                                  sem_b).wait()
            pltpu.sync_copy(rows_b, acc_s.at[idxs_v.at[jj + 1]], add=True)
            return carry

        lax.fori_loop(0, k // 2, body, 0)
        plsc.subcore_barrier()
        pltpu.sync_copy(acc_s.at[pl.ds(r0, rps)], out_hbm.at[c, pl.ds(r0, rps)])

    return kern


def _sc_prep(n_pad, d, k):
    rps = n_pad // NS

    @functools.partial(
        pl.kernel,
        out_type=(jax.ShapeDtypeStruct((NC, n_pad, d), F32),
                  jax.ShapeDtypeStruct((NC, n_pad, d), F32),
                  jax.ShapeDtypeStruct((NC, n_pad, d), F32)),
        mesh=_mesh(),
        compiler_params=_SC_PARAMS,
        scratch_types=[
            pltpu.VMEM_SHARED((n_pad, d), F32),
            pltpu.VMEM_SHARED((n_pad, d), F32),
            pltpu.VMEM_SHARED((n_pad, d), F32),
            pltpu.VMEM_SHARED((n_pad, d), F32),
            pltpu.VMEM((k, C), jnp.int32),
            pltpu.VMEM((k, C), jnp.int32),
            pltpu.VMEM((C, d), F32),
            pltpu.VMEM((C, d), F32),
            pltpu.SemaphoreType.DMA,
        ],
    )
    def kern(src_hbm, dst_hbm, t_hbm, zeros_hbm, outSt_hbm, outDi_hbm, outDo_hbm,
             t_s, accSt, accDi, accDo, src_v, dst_v, gt, ones_v, sem_p):
        c = lax.axis_index("c")
        s = lax.axis_index("s")
        wid = c * NS + s
        r0 = s * rps
        pltpu.sync_copy(t_hbm.at[pl.ds(r0, rps)], t_s.at[pl.ds(r0, rps)])
        pltpu.sync_copy(zeros_hbm.at[pl.ds(r0, rps)], accSt.at[pl.ds(r0, rps)])
        pltpu.sync_copy(zeros_hbm.at[pl.ds(r0, rps)], accDi.at[pl.ds(r0, rps)])
        pltpu.sync_copy(zeros_hbm.at[pl.ds(r0, rps)], accDo.at[pl.ds(r0, rps)])
        pltpu.sync_copy(src_hbm.at[pl.ds(wid * k, k)], src_v)
        pltpu.sync_copy(dst_hbm.at[pl.ds(wid * k, k)], dst_v)
        one = jnp.ones((L,), F32)
        for i in range(C):
            ones_v[i, :] = one
        plsc.subcore_barrier()

        def body(j, carry):
            cp = pltpu.async_copy(t_s.at[src_v.at[j]], gt, sem_p)
            pltpu.sync_copy(ones_v, accDi.at[dst_v.at[j]], add=True)
            pltpu.sync_copy(ones_v, accDo.at[src_v.at[j]], add=True)
            cp.wait()
            pltpu.sync_copy(gt, accSt.at[dst_v.at[j]], add=True)
            return carry

        lax.fori_loop(0, k, body, 0)
        plsc.subcore_barrier()
        pltpu.sync_copy(accSt.at[pl.ds(r0, rps)], outSt_hbm.at[c, pl.ds(r0, rps)])
        pltpu.sync_copy(accDi.at[pl.ds(r0, rps)], outDi_hbm.at[c, pl.ds(r0, rps)])
        pltpu.sync_copy(accDo.at[pl.ds(r0, rps)], outDo_hbm.at[c, pl.ds(r0, rps)])

    return kern


def _sc_edge(n_pad, d, k):
    rps = n_pad // NS

    @functools.partial(
        pl.kernel,
        out_type=jax.ShapeDtypeStruct((NC, n_pad, d), F32),
        mesh=_mesh(),
        compiler_params=_SC_PARAMS,
        scratch_types=[
            pltpu.VMEM_SHARED((n_pad, d), F32),
            pltpu.VMEM_SHARED((n_pad, d), F32),
            pltpu.VMEM_SHARED((n_pad, d), F32),
            pltpu.VMEM((k, C), jnp.int32),
            pltpu.VMEM((k, C), jnp.int32),
            pltpu.VMEM((2, C, d), F32),
            pltpu.VMEM((2, C, d), F32),
            pltpu.VMEM((2, C, d), F32),
            pltpu.VMEM((2, C, d), F32),
            pltpu.VMEM((C, d), F32),
            pltpu.VMEM((C, d), F32),
            pltpu.SemaphoreType.DMA,
            pltpu.SemaphoreType.DMA,
        ],
    )
    def kern(src_hbm, dst_hbm, h_hbm, m_hbm, zeros_hbm, out_hbm,
             h_s, m_s, acc_s, src_v, dst_v, ha, hb, ma, mb, pos_v, neg_v,
             sem_a, sem_b):
        c = lax.axis_index("c")
        s = lax.axis_index("s")
        wid = c * NS + s
        r0 = s * rps
        pltpu.sync_copy(h_hbm.at[pl.ds(r0, rps)], h_s.at[pl.ds(r0, rps)])
        pltpu.sync_copy(m_hbm.at[pl.ds(r0, rps)], m_s.at[pl.ds(r0, rps)])
        pltpu.sync_copy(zeros_hbm.at[pl.ds(r0, rps)], acc_s.at[pl.ds(r0, rps)])
        pltpu.sync_copy(src_hbm.at[pl.ds(wid * k, k)], src_v)
        pltpu.sync_copy(dst_hbm.at[pl.ds(wid * k, k)], dst_v)
        plsc.subcore_barrier()
        iota = lax.iota(jnp.int32, L)
        shuf = [lax.bitwise_xor(iota, jnp.int32(r)) for r in (1, 2, 4, 8)]
        dnums = lax.GatherDimensionNumbers(
            offset_dims=(), collapsed_slice_dims=(0,), start_index_map=(0,))

        def hsum(v):
            for idx in shuf:
                v = v + lax.gather(
                    v, idx[:, None], dnums, (1,),
                    mode=lax.GatherScatterMode.PROMISE_IN_BOUNDS)
            return v

        def fire(j, b, sem):
            pltpu.async_copy(h_s.at[src_v.at[j]], ha.at[b], sem)
            pltpu.async_copy(h_s.at[dst_v.at[j]], hb.at[b], sem)
            pltpu.async_copy(m_s.at[src_v.at[j]], ma.at[b], sem)
            pltpu.async_copy(m_s.at[dst_v.at[j]], mb.at[b], sem)

        def drain(j, b, sem):
            pltpu.make_async_copy(h_s.at[src_v.at[j]], ha.at[b], sem).wait()
            pltpu.make_async_copy(h_s.at[dst_v.at[j]], hb.at[b], sem).wait()
            pltpu.make_async_copy(m_s.at[src_v.at[j]], ma.at[b], sem).wait()
            pltpu.make_async_copy(m_s.at[dst_v.at[j]], mb.at[b], sem).wait()

        def process(j, b):
            @plsc.parallel_loop(0, C, step=1, unroll=8)
            def _edge(i):
                df = ha[b, i, :] - hb[b, i, :]
                d2 = hsum(df * df)
                cc = hsum(ma[b, i, :] * mb[b, i, :])
                yi = lax.bitcast_convert_type(
                    jnp.full((L,), 0x5F3759DF, jnp.int32)
                    - lax.shift_right_logical(
                        lax.bitcast_convert_type(d2, jnp.int32), 1),
                    F32)
                hd2 = 0.5 * d2
                yi = yi * (1.5 - hd2 * yi * yi)
                yi = yi * (1.5 - hd2 * yi * yi)
                coef = cc * yi * yi * yi
                v = coef * df
                pos_v[i, :] = v
                neg_v[i, :] = -v

            pltpu.sync_copy(pos_v, acc_s.at[dst_v.at[j]], add=True)
            pltpu.sync_copy(neg_v, acc_s.at[src_v.at[j]], add=True)

        fire(0, 0, sem_a)

        def body(j, carry):
            jj = 2 * j
            fire(jj + 1, 1, sem_b)
            drain(jj, 0, sem_a)
            process(jj, 0)

            @pl.when(jj + 2 < k)
            def _():
                fire(jj + 2, 0, sem_a)

            drain(jj + 1, 1, sem_b)
            process(jj + 1, 1)
            return carry

        lax.fori_loop(0, k // 2, body, 0)
        plsc.subcore_barrier()
        pltpu.sync_copy(acc_s.at[pl.ds(r0, rps)], out_hbm.at[c, pl.ds(r0, rps)])

    return kern


def _tc_call(body, out_shapes, *args):
    return pl.pallas_call(
        body,
        out_shape=tuple(jax.ShapeDtypeStruct(s, F32) for s in out_shapes),
    )(*args)


def _tc_pre(p128, m128):
    def body(p_ref, m_ref, t_ref):
        m = m_ref[...]
        valid = m > 0
        t_ref[...] = jnp.where(valid, p_ref[...] / jnp.where(valid, m, 1.0), 0.0)

    (t,) = _tc_call(body, [p128.shape], p128, m128)
    return t


def _tc1(accSt, accDi, accDo, q128, t128):
    def body(st_ref, di_ref, do_ref, q_ref, t_ref,
             rsin_ref, rsout_ref, x1_ref, dhdp_ref):
        rsin_ref[...] = lax.rsqrt(jnp.maximum(di_ref[0] + di_ref[1], 1.0))
        rsout_ref[...] = lax.rsqrt(jnp.maximum(do_ref[0] + do_ref[1], 1.0))
        x1_ref[...] = q_ref[...] * rsout_ref[...]
        dhdp_ref[...] = st_ref[0] + st_ref[1] + t_ref[...]

    return _tc_call(body, [q128.shape] * 4, accSt, accDi, accDo, q128, t128)


def _tc2(acc, rs_in, rs_out, W1big, b1big, W2big):
    r128, _ = rs_in.shape
    hidb = W1big.shape[1]

    def body(acc_ref, rsin_ref, rsout_ref, w1_ref, b1_ref, w2_ref,
             h2_ref, a_ref):
        xw = (acc_ref[0] + acc_ref[1]) * rsin_ref[...]
        a = jnp.dot(xw, w1_ref[...], preferred_element_type=F32) + b1_ref[...]
        a_ref[...] = a
        h2_ref[...] = jnp.dot(jnp.maximum(a, 0.0), w2_ref[...],
                              preferred_element_type=F32) * rsout_ref[...]

    return _tc_call(body, [rs_in.shape, (r128, hidb)],
                    acc, rs_in, rs_out, W1big, b1big, W2big)


def _tc3(acc, rs_in, q128, b2big):
    def body(acc_ref, rsin_ref, q_ref, b2_ref, h_ref):
        h_ref[...] = ((acc_ref[0] + acc_ref[1]) * rsin_ref[...]
                      + b2_ref[...] + q_ref[...])

    (h,) = _tc_call(body, [q128.shape], acc, rs_in, q128, b2big)
    return h


def _tc5(accG, rs_in, gravity):
    def body(acc_ref, rsin_ref, grav_ref, g_ref, y_ref):
        g = 0.5 * grav_ref[0, 0] * (acc_ref[0] + acc_ref[1])
        g_ref[...] = g
        y_ref[...] = g * rsin_ref[...]

    return _tc_call(body, [rs_in.shape] * 2, accG, rs_in, gravity)


def _tc6(acc, a_big, rs_in, rs_out, W2Tbig, W1Tbig):
    def body(acc_ref, a_ref, rsin_ref, rsout_ref, w2t_ref, w1t_ref, y2_ref):
        zw = (acc_ref[0] + acc_ref[1]) * rsout_ref[...]
        u = jnp.dot(zw, w2t_ref[...], preferred_element_type=F32)
        v = jnp.where(a_ref[...] > 0, u, 0.0)
        y2_ref[...] = jnp.dot(v, w1t_ref[...],
                              preferred_element_type=F32) * rsin_ref[...]

    (y2,) = _tc_call(body, [rs_in.shape], acc, a_big, rs_in, rs_out,
                     W2Tbig, W1Tbig)
    return y2


def _tc7(acc, rs_out, g128):
    def body(acc_ref, rsout_ref, g_ref, dhdq_ref):
        dhdq_ref[...] = (acc_ref[0] + acc_ref[1]) * rsout_ref[...] + g_ref[...]

    (dhdq,) = _tc_call(body, [rs_out.shape], acc, rs_out, g128)
    return dhdq


def kernel(q, p, edge_index, M, W1, b1, W2, b2, gravity):
    n, d = q.shape
    e = edge_index.shape[1]
    nw = NC * NS
    n_pad = -(-(n + 1) // (NS * 8)) * NS * 8
    k = -(-e // (nw * C))
    e_pad = nw * C * k
    r128 = n_pad * d // 128
    nb = 128 // d

    ei = jnp.pad(edge_index.astype(jnp.int32), ((0, 0), (0, e_pad - e)),
                 constant_values=n).reshape(2, e_pad // C, C)
    srcp, dstp = ei[0], ei[1]

    rv = n * d // 128
    rpad = ((0, r128 - rv), (0, 0))
    m = jnp.diagonal(M, axis1=1, axis2=2)
    q128 = jnp.pad(q.reshape(rv, 128), rpad)
    p128 = jnp.pad(p.reshape(rv, 128), rpad)
    m128 = jnp.pad(m.reshape(rv, 128), rpad)
    m_pad = m128.reshape(n_pad, d)
    zeros16 = jnp.zeros((n_pad, d), F32)

    eye = jnp.eye(nb, dtype=F32)
    W1big = jnp.kron(eye, W1)
    W2big = jnp.kron(eye, W2)
    W2Tbig = jnp.kron(eye, W2.T)
    W1Tbig = jnp.kron(eye, W1.T)
    b1big = jnp.tile(b1, nb).reshape(1, nb * b1.shape[0])
    b2big = jnp.tile(b2, nb).reshape(1, 128)

    def v128(acc):
        return acc.reshape(NC, r128, 128)

    def v16(x):
        return x.reshape(n_pad, d)

    sc_gs = _sc_gs(n_pad, d, k)

    t128 = _tc_pre(p128, m128)
    accSt, accDi, accDo = _sc_prep(n_pad, d, k)(srcp, dstp, v16(t128), zeros16)
    rs_in, rs_out, x1, dhdp = _tc1(v128(accSt), v128(accDi), v128(accDo),
                                   q128, t128)

    acc1 = sc_gs(srcp, dstp, v16(x1), zeros16)
    h2, a_big = _tc2(v128(acc1), rs_in, rs_out, W1big, b1big, W2big)
    acc2 = sc_gs(srcp, dstp, v16(h2), zeros16)
    h = _tc3(v128(acc2), rs_in, q128, b2big)

    accG = _sc_edge(n_pad, d, k)(srcp, dstp, v16(h), m_pad, zeros16)
    g128, y = _tc5(v128(accG), rs_in, gravity)

    accZ = sc_gs(dstp, srcp, v16(y), zeros16)
    y2 = _tc6(v128(accZ), a_big, rs_in, rs_out, W2Tbig, W1Tbig)
    accZ2 = sc_gs(dstp, srcp, v16(y2), zeros16)
    dhdq = _tc7(v128(accZ2), rs_out, g128)

    return jnp.concatenate([v16(dhdq)[:n], v16(dhdp)[:n]], axis=1)
